# baseline (device time: 153109 ns/iter reference)
import jax
import jax.numpy as jnp
from jax import lax
from jax.experimental import pallas as pl
from jax.experimental.pallas import tpu as pltpu

N_DEV = 4
R = 4


def kernel(x, w_mat):
    m_total, k_per = x.shape
    k_per2, n = w_mat.shape
    assert k_per == k_per2
    m_per = m_total // N_DEV
    half = n // 2
    sub = m_per // R

    def body(x_hbm, w_hbm, out_hbm,
             p0, recv_cw, recv_ccw, xc, wv, stage,
             cw_send_sems, cw_recv_sems, ccw_send_sems, ccw_recv_sems,
             load_sems, out_sems, credit_cw, credit_ccw):
        d = lax.axis_index("i")
        left = lax.rem(d + N_DEV - 1, N_DEV)
        right = lax.rem(d + 1, N_DEV)

        lcols = pl.ds(0, half)
        rcols = pl.ds(half, half)

        def rows(r):
            return pl.ds(r * sub, sub)

        XOFF = (N_DEV - 1, 1, 2, 0)
        coffs = [lax.rem(d + off, N_DEV) * m_per for off in XOFF]
        subloads = {}
        for j in (0, 1):
            for r in range(R):
                subloads[j, r] = pltpu.make_async_copy(
                    x_hbm.at[pl.ds(coffs[j] + r * sub, sub), :],
                    xc.at[j, rows(r)],
                    load_sems.at[j * R + r],
                )
        loads = {}
        for j in (2, 3):
            loads[j] = pltpu.make_async_copy(
                x_hbm.at[pl.ds(coffs[j], m_per), :], xc.at[j],
                load_sems.at[2 * R + j - 2],
            )
        wloads = [
            pltpu.make_async_copy(
                w_hbm.at[:, pl.ds(h * half, half)], wv.at[:, pl.ds(h * half, half)],
                load_sems.at[2 * R + 2 + h],
            )
            for h in (0, 1)
        ]
        subloads[0, 0].start()
        wloads[0].start()
        subloads[1, 0].start()
        wloads[1].start()
        for r in range(1, R):
            subloads[0, r].start()
            subloads[1, r].start()
        loads[2].start()
        loads[3].start()

        def xdot(j, w_cols, r=None):
            off = 0 if r is None else r * sub
            return jnp.dot(
                xc[j, pl.ds(off, m_per if r is None else sub), :],
                wv[:, w_cols],
                preferred_element_type=jnp.float32,
            )

        def mk(s, r, src, dst, send_sems, recv_sems, tgt):
            return pltpu.make_async_remote_copy(
                src_ref=src, dst_ref=dst,
                send_sem=send_sems.at[s * R + r],
                recv_sem=recv_sems.at[s * R + r],
                device_id=(tgt,), device_id_type=pl.DeviceIdType.MESH,
            )

        def mk_cw(s, r, src, dst):
            return mk(s, r, src, dst, cw_send_sems, cw_recv_sems, right)

        def mk_ccw(s, r, src, dst):
            return mk(s, r, src, dst, ccw_send_sems, ccw_recv_sems, left)

        barrier_sem = pltpu.get_barrier_semaphore()
        for nbr in (left, right):
            pl.semaphore_signal(
                barrier_sem, inc=1,
                device_id=(nbr,), device_id_type=pl.DeviceIdType.MESH,
            )

        cw0, ccw0 = [], []
        for r in range(R):
            subloads[0, r].wait()
            if r == 0:
                wloads[0].wait()
            p0[0, rows(r), :] = xdot(0, lcols, r)
            if r == 0:
                pl.semaphore_wait(barrier_sem, 2)
            cw0.append(mk_cw(0, r, p0.at[0, rows(r)], recv_cw.at[0, rows(r)]))
            cw0[r].start()
            subloads[1, r].wait()
            if r == 0:
                wloads[1].wait()
            p0[1, rows(r), :] = xdot(1, rcols, r)
            ccw0.append(mk_ccw(0, r, p0.at[1, rows(r)], recv_ccw.at[0, rows(r)]))
            ccw0[r].start()

        loads[2].wait()
        stage[:, lcols] = xdot(2, lcols)
        stage[:, rcols] = xdot(2, rcols)

        cw1, ccw1 = [], []
        for r in range(R):
            cw0[r].wait_recv()
            recv_cw[0, rows(r), :] = (
                recv_cw[0, rows(r), :] + stage[rows(r), lcols]
            )
            cw1.append(mk_cw(1, r, recv_cw.at[0, rows(r)], recv_cw.at[1, rows(r)]))
            cw1[r].start()
            ccw0[r].wait_recv()
            recv_ccw[0, rows(r), :] = (
                recv_ccw[0, rows(r), :] + stage[rows(r), rcols]
            )
            ccw1.append(mk_ccw(1, r, recv_ccw.at[0, rows(r)], recv_ccw.at[1, rows(r)]))
            ccw1[r].start()
        for r in range(R):
            cw0[r].wait_send()
            ccw0[r].wait_send()

        stage[:, lcols] = xdot(1, lcols)
        stage[:, rcols] = xdot(0, rcols)

        for r in range(R):
            cw1[r].wait_send()
            pl.semaphore_signal(
                credit_cw, inc=1,
                device_id=(left,), device_id_type=pl.DeviceIdType.MESH,
            )
            ccw1[r].wait_send()
            pl.semaphore_signal(
                credit_ccw, inc=1,
                device_id=(right,), device_id_type=pl.DeviceIdType.MESH,
            )

        cw2, ccw2 = [], []
        for r in range(R):
            cw1[r].wait_recv()
            recv_cw[1, rows(r), :] = (
                recv_cw[1, rows(r), :] + stage[rows(r), lcols]
            )
            pl.semaphore_wait(credit_cw, 1)
            cw2.append(mk_cw(2, r, recv_cw.at[1, rows(r)], recv_cw.at[0, rows(r)]))
            cw2[r].start()
            ccw1[r].wait_recv()
            recv_ccw[1, rows(r), :] = (
                recv_ccw[1, rows(r), :] + stage[rows(r), rcols]
            )
            pl.semaphore_wait(credit_ccw, 1)
            ccw2.append(mk_ccw(2, r, recv_ccw.at[1, rows(r)], recv_ccw.at[0, rows(r)]))
            ccw2[r].start()

        loads[3].wait()
        stage[:, lcols] = xdot(3, lcols)
        stage[:, rcols] = xdot(3, rcols)

        outs = []
        for r in range(R):
            cw2[r].wait_recv()
            stage[rows(r), lcols] = jnp.maximum(
                recv_cw[0, rows(r), :] + stage[rows(r), lcols], 0.0
            )
            ccw2[r].wait_recv()
            stage[rows(r), rcols] = jnp.maximum(
                recv_ccw[0, rows(r), :] + stage[rows(r), rcols], 0.0
            )
            outs.append(pltpu.make_async_copy(
                stage.at[rows(r), :], out_hbm.at[rows(r), :], out_sems.at[r],
            ))
            outs[r].start()
        for r in range(R):
            outs[r].wait()
            cw2[r].wait_send()
            ccw2[r].wait_send()

    return pl.pallas_call(
        body,
        out_shape=jax.ShapeDtypeStruct((m_per, n), jnp.float32),
        in_specs=[
            pl.BlockSpec(memory_space=pltpu.MemorySpace.HBM),
            pl.BlockSpec(memory_space=pltpu.MemorySpace.HBM),
        ],
        out_specs=pl.BlockSpec(memory_space=pltpu.MemorySpace.HBM),
        scratch_shapes=[
            pltpu.VMEM((2, m_per, half), jnp.float32),
            pltpu.VMEM((2, m_per, half), jnp.float32),
            pltpu.VMEM((2, m_per, half), jnp.float32),
            pltpu.VMEM((N_DEV, m_per, k_per), jnp.float32),
            pltpu.VMEM((k_per, n), jnp.float32),
            pltpu.VMEM((m_per, n), jnp.float32),
            pltpu.SemaphoreType.DMA(((N_DEV - 1) * R,)),
            pltpu.SemaphoreType.DMA(((N_DEV - 1) * R,)),
            pltpu.SemaphoreType.DMA(((N_DEV - 1) * R,)),
            pltpu.SemaphoreType.DMA(((N_DEV - 1) * R,)),
            pltpu.SemaphoreType.DMA((2 * R + 4,)),
            pltpu.SemaphoreType.DMA((R,)),
            pltpu.SemaphoreType.REGULAR,
            pltpu.SemaphoreType.REGULAR,
        ],
        compiler_params=pltpu.CompilerParams(
            collective_id=0,
            vmem_limit_bytes=128 * 1024 * 1024,
        ),
    )(x, w_mat)
